# Initial kernel scaffold; baseline (speedup 1.0000x reference)
import functools

import jax
import jax.numpy as jnp
from jax import lax
from jax.experimental import pallas as pl
from jax.experimental.pallas import tpu as pltpu
from jax.experimental.pallas import tpu_sc as plsc

N = 10000
NP = 10240
D = 128
E = 320000
NC = 2
NS = 16
NW = NC * NS
EW = E // NW
K = 80
CH = EW // K
RPT = NP // NS
WS = 16


def _sc_mesh():
    return plsc.VectorSubcoreMesh(core_axis_name="c", subcore_axis_name="s")


@functools.partial(
    pl.kernel,
    out_type=jax.ShapeDtypeStruct((NC, NP, D), jnp.float32),
    mesh=_sc_mesh(),
    scratch_types=[
        pltpu.VMEM((K,), jnp.int32),
        pltpu.VMEM((K,), jnp.int32),
        pltpu.VMEM((K, D), jnp.float32),
        pltpu.VMEM_SHARED((NP, D), jnp.float32),
        pltpu.SemaphoreType.DMA,
    ],
)
def _row_agg(y_hbm, src_hbm, dst_hbm, zrow_hbm, out_hbm, si, di, rows, acc, sem):
    cid = lax.axis_index("c")
    sid = lax.axis_index("s")
    wid = sid * NC + cid
    pltpu.sync_copy(zrow_hbm, acc.at[pl.ds(sid * RPT, RPT)])
    plsc.subcore_barrier()

    def body(i, carry):
        off = pl.multiple_of(wid * EW + i * K, 8)
        pltpu.sync_copy(src_hbm.at[pl.ds(off, K)], si)
        pltpu.sync_copy(dst_hbm.at[pl.ds(off, K)], di)
        pltpu.async_copy(y_hbm.at[si], rows, sem).wait()
        pltpu.sync_copy(rows, acc.at[di], add=True)
        return carry

    lax.fori_loop(0, CH, body, 0)
    plsc.subcore_barrier()
    pltpu.sync_copy(acc.at[pl.ds(sid * RPT, RPT)],
                    out_hbm.at[cid, pl.ds(sid * RPT, RPT)])


@functools.partial(
    pl.kernel,
    out_type=jax.ShapeDtypeStruct((NC, NP, WS), jnp.float32),
    mesh=_sc_mesh(),
    scratch_types=[
        pltpu.VMEM((K,), jnp.int32),
        pltpu.VMEM((K,), jnp.int32),
        pltpu.VMEM((K, WS), jnp.float32),
        pltpu.VMEM_SHARED((NP, WS), jnp.float32),
        pltpu.SemaphoreType.DMA,
    ],
)
def _scalar_agg(tab_hbm, src_hbm, dst_hbm, zrow_hbm, out_hbm, si, di, vals, acc, sem):
    cid = lax.axis_index("c")
    sid = lax.axis_index("s")
    wid = sid * NC + cid
    pltpu.sync_copy(zrow_hbm, acc.at[pl.ds(sid * RPT, RPT)])
    plsc.subcore_barrier()

    def body(i, carry):
        off = pl.multiple_of(wid * EW + i * K, 8)
        pltpu.sync_copy(src_hbm.at[pl.ds(off, K)], si)
        pltpu.sync_copy(dst_hbm.at[pl.ds(off, K)], di)
        pltpu.async_copy(tab_hbm.at[si], vals, sem).wait()
        pltpu.sync_copy(vals, acc.at[di], add=True)
        return carry

    lax.fori_loop(0, CH, body, 0)
    plsc.subcore_barrier()
    pltpu.sync_copy(acc.at[pl.ds(sid * RPT, RPT)],
                    out_hbm.at[cid, pl.ds(sid * RPT, RPT)])


@functools.partial(
    pl.kernel,
    out_type=jax.ShapeDtypeStruct((NC, NP, WS), jnp.float32),
    mesh=_sc_mesh(),
    scratch_types=[
        pltpu.VMEM((K,), jnp.int32),
        pltpu.VMEM((K, WS), jnp.float32),
        pltpu.VMEM_SHARED((NP, WS), jnp.float32),
    ],
)
def _deg_agg(ones_hbm, dst_hbm, zrow_hbm, out_hbm, di, vals, acc):
    cid = lax.axis_index("c")
    sid = lax.axis_index("s")
    wid = sid * NC + cid
    pltpu.sync_copy(ones_hbm, vals)
    pltpu.sync_copy(zrow_hbm, acc.at[pl.ds(sid * RPT, RPT)])
    plsc.subcore_barrier()

    def body(i, carry):
        off = pl.multiple_of(wid * EW + i * K, 8)
        pltpu.sync_copy(dst_hbm.at[pl.ds(off, K)], di)
        pltpu.sync_copy(vals, acc.at[di], add=True)
        return carry

    lax.fori_loop(0, CH, body, 0)
    plsc.subcore_barrier()
    pltpu.sync_copy(acc.at[pl.ds(sid * RPT, RPT)],
                    out_hbm.at[cid, pl.ds(sid * RPT, RPT)])


def _tc_b_body(deg0, deg1, x, w1, y_out, dinv_out):
    dinv = lax.rsqrt(deg0[...] + deg1[...] + 1.0)
    xw = jnp.dot(x[...], w1[...], preferred_element_type=jnp.float32)
    y_out[...] = xw * dinv
    dinv_out[...] = dinv


_tc_b = pl.pallas_call(
    _tc_b_body,
    out_shape=[
        jax.ShapeDtypeStruct((N, D), jnp.float32),
        jax.ShapeDtypeStruct((N, 1), jnp.float32),
    ],
)


def _tc_d_body(agg0, agg1, y, dinv, b1, w2, t_out):
    h = jnp.maximum((agg0[...] + agg1[...] + y[...]) * dinv[...] + b1[...], 0.0)
    s = jnp.dot(h, w2[...], preferred_element_type=jnp.float32)
    t_out[...] = jnp.broadcast_to(s * dinv[...], (N, WS))


_tc_d = pl.pallas_call(
    _tc_d_body,
    out_shape=jax.ShapeDtypeStruct((N, WS), jnp.float32),
)


def _tc_f_body(acc0, acc1, t, dinv, b2, out):
    o = jax.nn.sigmoid((acc0[...] + acc1[...] + t[...]) * dinv[...] + b2[...])
    out[...] = (jnp.sum(o) / N).reshape(1, 1)


_tc_f = pl.pallas_call(
    _tc_f_body,
    out_shape=jax.ShapeDtypeStruct((1, 1), jnp.float32),
)


def kernel(x, edge_index, W1, b1, W2, b2):
    src = edge_index[0].astype(jnp.int32)
    dst = edge_index[1].astype(jnp.int32)
    f32 = jnp.float32

    onesK = jnp.ones((K, WS), f32)
    zrow_d = jnp.zeros((RPT, D), f32)
    zrow_w = jnp.zeros((RPT, WS), f32)

    degp = _deg_agg(onesK, dst, zrow_w)
    y, dinv = _tc_b(degp[0, :N, 0:1], degp[1, :N, 0:1], x, W1)

    aggp = _row_agg(y, src, dst, zrow_d)
    t16 = _tc_d(aggp[0, :N], aggp[1, :N], y, dinv,
                b1.reshape(1, D), W2)

    accp = _scalar_agg(t16, src, dst, zrow_w)
    out = _tc_f(accp[0, :N, 0:1], accp[1, :N, 0:1],
                t16[:, 0:1], dinv, b2.reshape(1, 1))
    return out.reshape(1)


# speedup vs baseline: 15.2567x; 15.2567x over previous
import functools

import jax
import jax.numpy as jnp
from jax import lax
from jax.experimental import pallas as pl
from jax.experimental.pallas import tpu as pltpu
from jax.experimental.pallas import tpu_sc as plsc

N = 10000
NP = 10240
D = 128
E = 320000
NC = 2
NS = 16
NW = NC * NS
EW = E // NW
K = 80
CH = EW // K
RPT = NP // NS
WS = 16


def _sc_mesh():
    return plsc.VectorSubcoreMesh(core_axis_name="c", subcore_axis_name="s",
                                  num_cores=NC, num_subcores=NS)


@functools.partial(
    pl.kernel,
    out_type=jax.ShapeDtypeStruct((NC, NP, D), jnp.float32),
    mesh=_sc_mesh(),
    scratch_types=[
        pltpu.VMEM((K,), jnp.int32),
        pltpu.VMEM((K,), jnp.int32),
        pltpu.VMEM((K, D), jnp.float32),
        pltpu.VMEM_SHARED((NP, D), jnp.float32),
        pltpu.SemaphoreType.DMA,
    ],
)
def _row_agg(y_hbm, src_hbm, dst_hbm, zrow_hbm, out_hbm, si, di, rows, acc, sem):
    cid = lax.axis_index("c")
    sid = lax.axis_index("s")
    wid = sid * NC + cid
    pltpu.sync_copy(zrow_hbm, acc.at[pl.ds(sid * RPT, RPT)])
    plsc.subcore_barrier()

    def body(i, carry):
        off = pl.multiple_of(wid * EW + i * K, 8)
        pltpu.sync_copy(src_hbm.at[pl.ds(off, K)], si)
        pltpu.sync_copy(dst_hbm.at[pl.ds(off, K)], di)
        pltpu.async_copy(y_hbm.at[si], rows, sem).wait()
        pltpu.sync_copy(rows, acc.at[di], add=True)
        return carry

    lax.fori_loop(0, CH, body, 0)
    plsc.subcore_barrier()
    pltpu.sync_copy(acc.at[pl.ds(sid * RPT, RPT)],
                    out_hbm.at[cid, pl.ds(sid * RPT, RPT)])


@functools.partial(
    pl.kernel,
    out_type=jax.ShapeDtypeStruct((NC, NP, WS), jnp.float32),
    mesh=_sc_mesh(),
    scratch_types=[
        pltpu.VMEM((K,), jnp.int32),
        pltpu.VMEM((K,), jnp.int32),
        pltpu.VMEM((K, WS), jnp.float32),
        pltpu.VMEM_SHARED((NP, WS), jnp.float32),
        pltpu.SemaphoreType.DMA,
    ],
    compiler_params=pltpu.CompilerParams(use_tc_tiling_on_sc=False),
)
def _scalar_agg(tab_hbm, src_hbm, dst_hbm, zrow_hbm, out_hbm, si, di, vals, acc, sem):
    cid = lax.axis_index("c")
    sid = lax.axis_index("s")
    wid = sid * NC + cid
    pltpu.sync_copy(zrow_hbm, acc.at[pl.ds(sid * RPT, RPT)])
    plsc.subcore_barrier()

    def body(i, carry):
        off = pl.multiple_of(wid * EW + i * K, 8)
        pltpu.sync_copy(src_hbm.at[pl.ds(off, K)], si)
        pltpu.sync_copy(dst_hbm.at[pl.ds(off, K)], di)
        pltpu.async_copy(tab_hbm.at[si], vals, sem).wait()
        pltpu.sync_copy(vals, acc.at[di], add=True)
        return carry

    lax.fori_loop(0, CH, body, 0)
    plsc.subcore_barrier()
    pltpu.sync_copy(acc.at[pl.ds(sid * RPT, RPT)],
                    out_hbm.at[cid, pl.ds(sid * RPT, RPT)])


@functools.partial(
    pl.kernel,
    out_type=jax.ShapeDtypeStruct((NC, NP, WS), jnp.float32),
    mesh=_sc_mesh(),
    scratch_types=[
        pltpu.VMEM((K,), jnp.int32),
        pltpu.VMEM((K, WS), jnp.float32),
        pltpu.VMEM_SHARED((NP, WS), jnp.float32),
    ],
    compiler_params=pltpu.CompilerParams(use_tc_tiling_on_sc=False),
)
def _deg_agg(ones_hbm, dst_hbm, zrow_hbm, out_hbm, di, vals, acc):
    cid = lax.axis_index("c")
    sid = lax.axis_index("s")
    wid = sid * NC + cid
    pltpu.sync_copy(ones_hbm, vals)
    pltpu.sync_copy(zrow_hbm, acc.at[pl.ds(sid * RPT, RPT)])
    plsc.subcore_barrier()

    def body(i, carry):
        off = pl.multiple_of(wid * EW + i * K, 8)
        pltpu.sync_copy(dst_hbm.at[pl.ds(off, K)], di)
        pltpu.sync_copy(vals, acc.at[di], add=True)
        return carry

    lax.fori_loop(0, CH, body, 0)
    plsc.subcore_barrier()
    pltpu.sync_copy(acc.at[pl.ds(sid * RPT, RPT)],
                    out_hbm.at[cid, pl.ds(sid * RPT, RPT)])


def _tc_b_body(deg0, deg1, x, w1, y_out, dinv_out):
    dinv = lax.rsqrt(deg0[...] + deg1[...] + 1.0)
    xw = jnp.dot(x[...], w1[...], preferred_element_type=jnp.float32)
    y_out[...] = xw * dinv
    dinv_out[...] = dinv


_tc_b = pl.pallas_call(
    _tc_b_body,
    out_shape=[
        jax.ShapeDtypeStruct((N, D), jnp.float32),
        jax.ShapeDtypeStruct((N, 1), jnp.float32),
    ],
)


def _tc_d_body(agg0, agg1, y, dinv, b1, w2, t_out):
    h = jnp.maximum((agg0[...] + agg1[...] + y[...]) * dinv[...] + b1[...], 0.0)
    s = jnp.dot(h, w2[...], preferred_element_type=jnp.float32)
    t_out[...] = jnp.broadcast_to(s * dinv[...], (N, WS))


_tc_d = pl.pallas_call(
    _tc_d_body,
    out_shape=jax.ShapeDtypeStruct((N, WS), jnp.float32),
)


def _tc_f_body(acc0, acc1, t, dinv, b2, out):
    o = jax.nn.sigmoid((acc0[...] + acc1[...] + t[...]) * dinv[...] + b2[...])
    out[...] = (jnp.sum(o) / N).reshape(1, 1)


_tc_f = pl.pallas_call(
    _tc_f_body,
    out_shape=jax.ShapeDtypeStruct((1, 1), jnp.float32),
)


def kernel(x, edge_index, W1, b1, W2, b2):
    src = edge_index[0].astype(jnp.int32)
    dst = edge_index[1].astype(jnp.int32)
    f32 = jnp.float32

    onesK = jnp.ones((K, WS), f32)
    zrow_d = jnp.zeros((RPT, D), f32)
    zrow_w = jnp.zeros((RPT, WS), f32)

    degp = _deg_agg(onesK, dst, zrow_w)
    y, dinv = _tc_b(degp[0, :N, 0:1], degp[1, :N, 0:1], x, W1)

    aggp = _row_agg(y, src, dst, zrow_d)
    t16 = _tc_d(aggp[0, :N], aggp[1, :N], y, dinv,
                b1.reshape(1, D), W2)

    accp = _scalar_agg(t16, src, dst, zrow_w)
    out = _tc_f(accp[0, :N, 0:1], accp[1, :N, 0:1],
                t16[:, 0:1], dinv, b2.reshape(1, 1))
    return out.reshape(1)


# speedup vs baseline: 33.7555x; 2.2125x over previous
import functools

import jax
import jax.numpy as jnp
from jax import lax
from jax.experimental import pallas as pl
from jax.experimental.pallas import tpu as pltpu
from jax.experimental.pallas import tpu_sc as plsc

N = 10000
NP = 10240
D = 128
E = 320000
NC = 2
NS = 16
NW = NC * NS
K = 128
CH = 80
EP = NW * CH * K
RPT = NP // NS
WS = 16
HJ = CH // 2


def _sc_mesh():
    return plsc.VectorSubcoreMesh(core_axis_name="c", subcore_axis_name="s",
                                  num_cores=NC, num_subcores=NS)


def _edge_pipeline(tab_hbm, acc, si2, di2, r0, r1, sg0, sg1, nchunks):
    half = nchunks // 2

    def gather(c, buf, sem):
        pltpu.async_copy(tab_hbm.at[si2.at[c]], buf, sem)

    def gather_wait(c, buf, sem):
        pltpu.make_async_copy(tab_hbm.at[si2.at[c]], buf, sem).wait()

    gather(0, r0, sg0)
    gather(1, r1, sg1)

    def body(j, carry):
        c0 = 2 * j
        c1 = 2 * j + 1
        gather_wait(c0, r0, sg0)
        s0 = pltpu.async_copy(r0, acc.at[di2.at[c0]], sg0, add=True)
        gather_wait(c1, r1, sg1)
        s1 = pltpu.async_copy(r1, acc.at[di2.at[c1]], sg1, add=True)
        s0.wait()

        @pl.when(j < half - 1)
        def _():
            gather(c0 + 2, r0, sg0)

        s1.wait()

        @pl.when(j < half - 1)
        def _():
            gather(c1 + 2, r1, sg1)

        return carry

    lax.fori_loop(0, half, body, 0)


@functools.partial(
    pl.kernel,
    out_type=jax.ShapeDtypeStruct((NC, NP, D), jnp.float32),
    mesh=_sc_mesh(),
    scratch_types=[
        pltpu.VMEM((CH // 2, K), jnp.int32),
        pltpu.VMEM((CH // 2, K), jnp.int32),
        pltpu.VMEM((K, D), jnp.float32),
        pltpu.VMEM((K, D), jnp.float32),
        pltpu.VMEM_SHARED((NP, D), jnp.float32),
        pltpu.SemaphoreType.DMA,
        pltpu.SemaphoreType.DMA,
    ],
)
def _row_agg(y_hbm, src_hbm, dst_hbm, zrow_hbm, out_hbm,
             si2, di2, r0, r1, acc, sg0, sg1):
    cid = lax.axis_index("c")
    sid = lax.axis_index("s")
    wid = sid * NC + cid
    pltpu.sync_copy(zrow_hbm, acc.at[pl.ds(sid * RPT, RPT)])
    plsc.subcore_barrier()
    for p in range(2):
        pltpu.sync_copy(src_hbm.at[wid, pl.ds(p * (CH // 2), CH // 2)], si2)
        pltpu.sync_copy(dst_hbm.at[wid, pl.ds(p * (CH // 2), CH // 2)], di2)
        _edge_pipeline(y_hbm, acc, si2, di2, r0, r1, sg0, sg1, CH // 2)
    plsc.subcore_barrier()
    pltpu.sync_copy(acc.at[pl.ds(sid * RPT, RPT)],
                    out_hbm.at[cid, pl.ds(sid * RPT, RPT)])


@functools.partial(
    pl.kernel,
    out_type=jax.ShapeDtypeStruct((NC, NP, WS), jnp.float32),
    mesh=_sc_mesh(),
    scratch_types=[
        pltpu.VMEM((CH, K), jnp.int32),
        pltpu.VMEM((CH, K), jnp.int32),
        pltpu.VMEM((K, WS), jnp.float32),
        pltpu.VMEM((K, WS), jnp.float32),
        pltpu.VMEM_SHARED((NP, WS), jnp.float32),
        pltpu.SemaphoreType.DMA,
        pltpu.SemaphoreType.DMA,
    ],
    compiler_params=pltpu.CompilerParams(use_tc_tiling_on_sc=False),
)
def _scalar_agg(tab_hbm, src_hbm, dst_hbm, zrow_hbm, out_hbm,
                si2, di2, r0, r1, acc, sg0, sg1):
    cid = lax.axis_index("c")
    sid = lax.axis_index("s")
    wid = sid * NC + cid
    pltpu.sync_copy(src_hbm.at[wid], si2)
    pltpu.sync_copy(dst_hbm.at[wid], di2)
    pltpu.sync_copy(zrow_hbm, acc.at[pl.ds(sid * RPT, RPT)])
    plsc.subcore_barrier()
    _edge_pipeline(tab_hbm, acc, si2, di2, r0, r1, sg0, sg1, CH)
    plsc.subcore_barrier()
    pltpu.sync_copy(acc.at[pl.ds(sid * RPT, RPT)],
                    out_hbm.at[cid, pl.ds(sid * RPT, RPT)])


@functools.partial(
    pl.kernel,
    out_type=jax.ShapeDtypeStruct((NC, NP, WS), jnp.float32),
    mesh=_sc_mesh(),
    scratch_types=[
        pltpu.VMEM((CH, K), jnp.int32),
        pltpu.VMEM((K, WS), jnp.float32),
        pltpu.VMEM_SHARED((NP, WS), jnp.float32),
        pltpu.SemaphoreType.DMA,
    ],
    compiler_params=pltpu.CompilerParams(use_tc_tiling_on_sc=False),
)
def _deg_agg(ones_hbm, dst_hbm, zrow_hbm, out_hbm, di2, vals, acc, sem):
    cid = lax.axis_index("c")
    sid = lax.axis_index("s")
    wid = sid * NC + cid
    pltpu.sync_copy(ones_hbm, vals)
    pltpu.sync_copy(dst_hbm.at[wid], di2)
    pltpu.sync_copy(zrow_hbm, acc.at[pl.ds(sid * RPT, RPT)])
    plsc.subcore_barrier()

    def fire(c, carry):
        pltpu.async_copy(vals, acc.at[di2.at[c]], sem, add=True)
        return carry

    lax.fori_loop(0, CH, fire, 0)

    def drain(c, carry):
        pltpu.make_async_copy(vals, acc.at[di2.at[c]], sem).wait()
        return carry

    lax.fori_loop(0, CH, drain, 0)
    plsc.subcore_barrier()
    pltpu.sync_copy(acc.at[pl.ds(sid * RPT, RPT)],
                    out_hbm.at[cid, pl.ds(sid * RPT, RPT)])


def _tc_b_body(deg0, deg1, x, w1, y_out, dinv_out):
    dinv = lax.rsqrt(deg0[...] + deg1[...] + 1.0)
    xw = jnp.dot(x[...], w1[...], preferred_element_type=jnp.float32)
    y_out[...] = xw * dinv
    dinv_out[...] = dinv


_tc_b = pl.pallas_call(
    _tc_b_body,
    out_shape=[
        jax.ShapeDtypeStruct((N, D), jnp.float32),
        jax.ShapeDtypeStruct((N, 1), jnp.float32),
    ],
)


def _tc_d_body(agg0, agg1, y, dinv, b1, w2, t_out):
    h = jnp.maximum((agg0[...] + agg1[...] + y[...]) * dinv[...] + b1[...], 0.0)
    s = jnp.dot(h, w2[...], preferred_element_type=jnp.float32)
    t_out[...] = jnp.broadcast_to(s * dinv[...], (N, WS))


_tc_d = pl.pallas_call(
    _tc_d_body,
    out_shape=jax.ShapeDtypeStruct((N, WS), jnp.float32),
)


def _tc_f_body(acc0, acc1, t, dinv, b2, out):
    o = jax.nn.sigmoid((acc0[...] + acc1[...] + t[...]) * dinv[...] + b2[...])
    out[...] = (jnp.sum(o) / N).reshape(1, 1)


_tc_f = pl.pallas_call(
    _tc_f_body,
    out_shape=jax.ShapeDtypeStruct((1, 1), jnp.float32),
)


def kernel(x, edge_index, W1, b1, W2, b2):
    f32 = jnp.float32
    src = edge_index[0].astype(jnp.int32)
    dst = edge_index[1].astype(jnp.int32)

    pad = jnp.arange(EP - E, dtype=jnp.int32)
    src3 = jnp.concatenate([src, pad % N]).reshape(NW, CH, K)
    dst3 = jnp.concatenate([dst, N + pad % (NP - N)]).reshape(NW, CH, K)

    onesK = jnp.ones((K, WS), f32)
    zrow_d = jnp.zeros((RPT, D), f32)
    zrow_w = jnp.zeros((RPT, WS), f32)

    degp = _deg_agg(onesK, dst3, zrow_w)
    y, dinv = _tc_b(degp[0, :N, 0:1], degp[1, :N, 0:1], x, W1)

    aggp = _row_agg(y, src3, dst3, zrow_d)
    t16 = _tc_d(aggp[0, :N], aggp[1, :N], y, dinv,
                b1.reshape(1, D), W2)

    accp = _scalar_agg(t16, src3, dst3, zrow_w)
    out = _tc_f(accp[0, :N, 0:1], accp[1, :N, 0:1],
                t16[:, 0:1], dinv, b2.reshape(1, 1))
    return out.reshape(1)


# speedup vs baseline: 35.0309x; 1.0378x over previous
import functools

import jax
import jax.numpy as jnp
from jax import lax
from jax.experimental import pallas as pl
from jax.experimental.pallas import tpu as pltpu
from jax.experimental.pallas import tpu_sc as plsc

N = 10000
NP = 10240
D = 128
E = 320000
NC = 2
NS = 16
NW = NC * NS
K = 128
CH = 80
EP = NW * CH * K
RPT = NP // NS
WS = 16
KS = 256
CHS = CH * K // KS


def _sc_mesh():
    return plsc.VectorSubcoreMesh(core_axis_name="c", subcore_axis_name="s",
                                  num_cores=NC, num_subcores=NS)


def _edge_pipeline(tab_hbm, acc, si2, di2, r0, r1, sg0, sg1, nchunks):
    half = nchunks // 2

    def gather(c, buf, sem):
        pltpu.async_copy(tab_hbm.at[si2.at[c]], buf, sem)

    def gather_wait(c, buf, sem):
        pltpu.make_async_copy(tab_hbm.at[si2.at[c]], buf, sem).wait()

    gather(0, r0, sg0)
    gather(1, r1, sg1)

    def body(j, carry):
        c0 = 2 * j
        c1 = 2 * j + 1
        gather_wait(c0, r0, sg0)
        s0 = pltpu.async_copy(r0, acc.at[di2.at[c0]], sg0, add=True)
        gather_wait(c1, r1, sg1)
        s1 = pltpu.async_copy(r1, acc.at[di2.at[c1]], sg1, add=True)
        s0.wait()

        @pl.when(j < half - 1)
        def _():
            gather(c0 + 2, r0, sg0)

        s1.wait()

        @pl.when(j < half - 1)
        def _():
            gather(c1 + 2, r1, sg1)

        return carry

    lax.fori_loop(0, half, body, 0)


@functools.partial(
    pl.kernel,
    out_type=jax.ShapeDtypeStruct((NC, NP, D), jnp.float32),
    mesh=_sc_mesh(),
    scratch_types=[
        pltpu.VMEM((CH // 2, K), jnp.int32),
        pltpu.VMEM((CH // 2, K), jnp.int32),
        pltpu.VMEM((K, D), jnp.float32),
        pltpu.VMEM((K, D), jnp.float32),
        pltpu.VMEM_SHARED((NP, D), jnp.float32),
        pltpu.SemaphoreType.DMA,
        pltpu.SemaphoreType.DMA,
    ],
)
def _row_agg(y_hbm, src_hbm, dst_hbm, zrow_hbm, out_hbm,
             si2, di2, r0, r1, acc, sg0, sg1):
    cid = lax.axis_index("c")
    sid = lax.axis_index("s")
    wid = sid * NC + cid
    pltpu.sync_copy(zrow_hbm, acc.at[pl.ds(sid * RPT, RPT)])
    plsc.subcore_barrier()
    for p in range(2):
        pltpu.sync_copy(src_hbm.at[wid, pl.ds(p * (CH // 2), CH // 2)], si2)
        pltpu.sync_copy(dst_hbm.at[wid, pl.ds(p * (CH // 2), CH // 2)], di2)
        _edge_pipeline(y_hbm, acc, si2, di2, r0, r1, sg0, sg1, CH // 2)
    plsc.subcore_barrier()
    pltpu.sync_copy(acc.at[pl.ds(sid * RPT, RPT)],
                    out_hbm.at[cid, pl.ds(sid * RPT, RPT)])


@functools.partial(
    pl.kernel,
    out_type=jax.ShapeDtypeStruct((NC, NP, WS), jnp.float32),
    mesh=_sc_mesh(),
    scratch_types=[
        pltpu.VMEM((CHS, KS), jnp.int32),
        pltpu.VMEM((CHS, KS), jnp.int32),
        pltpu.VMEM((KS, WS), jnp.float32),
        pltpu.VMEM((KS, WS), jnp.float32),
        pltpu.VMEM_SHARED((NP, WS), jnp.float32),
        pltpu.SemaphoreType.DMA,
        pltpu.SemaphoreType.DMA,
    ],
    compiler_params=pltpu.CompilerParams(use_tc_tiling_on_sc=False),
)
def _scalar_agg(tab_hbm, src_hbm, dst_hbm, zrow_hbm, out_hbm,
                si2, di2, r0, r1, acc, sg0, sg1):
    cid = lax.axis_index("c")
    sid = lax.axis_index("s")
    wid = sid * NC + cid
    pltpu.sync_copy(src_hbm.at[wid], si2)
    pltpu.sync_copy(dst_hbm.at[wid], di2)
    pltpu.sync_copy(zrow_hbm, acc.at[pl.ds(sid * RPT, RPT)])
    plsc.subcore_barrier()
    _edge_pipeline(tab_hbm, acc, si2, di2, r0, r1, sg0, sg1, CHS)
    plsc.subcore_barrier()
    pltpu.sync_copy(acc.at[pl.ds(sid * RPT, RPT)],
                    out_hbm.at[cid, pl.ds(sid * RPT, RPT)])


@functools.partial(
    pl.kernel,
    out_type=jax.ShapeDtypeStruct((NC, NP, WS), jnp.float32),
    mesh=_sc_mesh(),
    scratch_types=[
        pltpu.VMEM((CH, K), jnp.int32),
        pltpu.VMEM((K, WS), jnp.float32),
        pltpu.VMEM_SHARED((NP, WS), jnp.float32),
        pltpu.SemaphoreType.DMA,
    ],
    compiler_params=pltpu.CompilerParams(use_tc_tiling_on_sc=False),
)
def _deg_agg(ones_hbm, dst_hbm, zrow_hbm, out_hbm, di2, vals, acc, sem):
    cid = lax.axis_index("c")
    sid = lax.axis_index("s")
    wid = sid * NC + cid
    pltpu.sync_copy(ones_hbm, vals)
    pltpu.sync_copy(dst_hbm.at[wid], di2)
    pltpu.sync_copy(zrow_hbm, acc.at[pl.ds(sid * RPT, RPT)])
    plsc.subcore_barrier()

    def fire(c, carry):
        pltpu.async_copy(vals, acc.at[di2.at[c]], sem, add=True)
        return carry

    lax.fori_loop(0, CH, fire, 0)

    def drain(c, carry):
        pltpu.make_async_copy(vals, acc.at[di2.at[c]], sem).wait()
        return carry

    lax.fori_loop(0, CH, drain, 0)
    plsc.subcore_barrier()
    pltpu.sync_copy(acc.at[pl.ds(sid * RPT, RPT)],
                    out_hbm.at[cid, pl.ds(sid * RPT, RPT)])


def _tc_b_body(deg0, deg1, x, w1, y_out, dinv_out):
    dinv = lax.rsqrt(deg0[...] + deg1[...] + 1.0)
    xw = jnp.dot(x[...], w1[...], preferred_element_type=jnp.float32)
    y_out[...] = xw * dinv
    dinv_out[...] = dinv


_tc_b = pl.pallas_call(
    _tc_b_body,
    out_shape=[
        jax.ShapeDtypeStruct((N, D), jnp.float32),
        jax.ShapeDtypeStruct((N, 1), jnp.float32),
    ],
)


def _tc_d_body(agg0, agg1, y, dinv, b1, w2, t_out):
    h = jnp.maximum((agg0[...] + agg1[...] + y[...]) * dinv[...] + b1[...], 0.0)
    s = jnp.dot(h, w2[...], preferred_element_type=jnp.float32)
    t_out[...] = jnp.broadcast_to(s * dinv[...], (N, WS))


_tc_d = pl.pallas_call(
    _tc_d_body,
    out_shape=jax.ShapeDtypeStruct((N, WS), jnp.float32),
)


def _tc_f_body(acc0, acc1, t, dinv, b2, out):
    o = jax.nn.sigmoid((acc0[...] + acc1[...] + t[...]) * dinv[...] + b2[...])
    out[...] = (jnp.sum(o) / N).reshape(1, 1)


_tc_f = pl.pallas_call(
    _tc_f_body,
    out_shape=jax.ShapeDtypeStruct((1, 1), jnp.float32),
)


def kernel(x, edge_index, W1, b1, W2, b2):
    f32 = jnp.float32
    src = edge_index[0].astype(jnp.int32)
    dst = edge_index[1].astype(jnp.int32)

    pad = jnp.arange(EP - E, dtype=jnp.int32)
    src_p = jnp.concatenate([src, pad % N])
    dst_p = jnp.concatenate([dst, N + pad % (NP - N)])
    src3 = src_p.reshape(NW, CH, K)
    dst3 = dst_p.reshape(NW, CH, K)
    srcS = src_p.reshape(NW, CHS, KS)
    dstS = dst_p.reshape(NW, CHS, KS)

    onesK = jnp.ones((K, WS), f32)
    zrow_d = jnp.zeros((RPT, D), f32)
    zrow_w = jnp.zeros((RPT, WS), f32)

    degp = _deg_agg(onesK, dst3, zrow_w)
    y, dinv = _tc_b(degp[0, :N, 0:1], degp[1, :N, 0:1], x, W1)

    aggp = _row_agg(y, src3, dst3, zrow_d)
    t16 = _tc_d(aggp[0, :N], aggp[1, :N], y, dinv,
                b1.reshape(1, D), W2)

    accp = _scalar_agg(t16, srcS, dstS, zrow_w)
    out = _tc_f(accp[0, :N, 0:1], accp[1, :N, 0:1],
                t16[:, 0:1], dinv, b2.reshape(1, 1))
    return out.reshape(1)
